# baseline (speedup 1.0000x reference)
import functools

import jax
import jax.numpy as jnp
from jax.experimental import pallas as pl
from jax.experimental.pallas import tpu as pltpu

_KNOCK = -1e30


def _gat_block_body(xr_ref, x_ref, xt_ref, w_ref, o_ref,
                    nrm_ref, s_ref, fbf_ref, *, n_pts, k, rows):
    t = pl.program_id(1)

    @pl.when(t == 0)
    def _precompute():
        xv = x_ref[0]
        nrm_ref[0, :] = jnp.sum(xv * xv, axis=1)
        f = jnp.where(xv >= 0.0, xv, 0.01 * xv)
        fb16 = f.astype(jnp.bfloat16)
        fbf_ref[...] = fb16
        wb = w_ref[0].astype(jnp.bfloat16).astype(jnp.float32)
        s_ref[0, :] = jnp.sum(fb16.astype(jnp.float32) * wb[None, :], axis=1)

    xr = xr_ref[0]
    xt = xt_ref[0]
    nr = nrm_ref[0, pl.ds(t * rows, rows)]
    s_self = s_ref[0, pl.ds(t * rows, rows)]
    sb = s_ref[0, :][None, :]

    dot = jax.lax.dot_general(xr, xt, (((1,), (0,)), ((), ())),
                              preferred_element_type=jnp.float32)
    pd = 2.0 * dot - nr[:, None] - nrm_ref[0, :][None, :]

    ncolf = -jax.lax.broadcasted_iota(jnp.int32, (rows, n_pts), 1).astype(jnp.float32)
    logits = []
    for _ in range(k):
        mx = jnp.max(pd, axis=1, keepdims=True)
        selneg = jnp.max(jnp.where(pd == mx, ncolf, -jnp.inf),
                         axis=1, keepdims=True)
        hit = ncolf == selneg
        logits.append(jnp.sum(jnp.where(hit, sb, 0.0), axis=1))
        pd = jnp.where(hit, _KNOCK, pd)
    logits.append(s_self)
    lg = jnp.stack(logits, axis=1)

    e = jnp.exp(lg - jnp.max(lg, axis=1, keepdims=True))
    a = e / jnp.sum(e, axis=1, keepdims=True)
    amp = jnp.sum(a.astype(jnp.bfloat16).astype(jnp.float32), axis=1)

    rowg = (jax.lax.broadcasted_iota(jnp.int32, (rows, n_pts), 0)
            + t * rows).astype(jnp.float32)
    grp = jnp.where(pd < -1e29, 1.0, 0.0) + jnp.where(-ncolf == rowg, 1.0, 0.0)
    ssum = jax.lax.dot_general(grp.astype(jnp.bfloat16), fbf_ref[...],
                               (((1,), (0,)), ((), ())),
                               preferred_element_type=jnp.float32)
    o_ref[0] = (amp[:, None] * ssum) / 9.0 + xr


def kernel(points, W_att):
    b, c, n = points.shape
    n_blocks = W_att.shape[0]
    k = 8
    rows = min(128, n)

    body = functools.partial(_gat_block_body, n_pts=n, k=k, rows=rows)
    call = pl.pallas_call(
        body,
        grid=(b, n // rows),
        in_specs=[
            pl.BlockSpec((1, rows, c), lambda i, t: (i, t, 0)),
            pl.BlockSpec((1, n, c), lambda i, t: (i, 0, 0)),
            pl.BlockSpec((1, c, n), lambda i, t: (i, 0, 0)),
            pl.BlockSpec((1, c), lambda i, t: (0, 0)),
        ],
        out_specs=pl.BlockSpec((1, rows, c), lambda i, t: (i, t, 0)),
        out_shape=jax.ShapeDtypeStruct((b, n, c), jnp.float32),
        scratch_shapes=[
            pltpu.VMEM((1, n), jnp.float32),
            pltpu.VMEM((1, n), jnp.float32),
            pltpu.VMEM((n, c), jnp.bfloat16),
        ],
    )

    x = jnp.swapaxes(points, 1, 2)
    for blk in range(n_blocks):
        xt = jnp.swapaxes(x, 1, 2)
        w = W_att[blk, :, 0][None, :]
        x = call(x, x, xt, w)
    return jnp.swapaxes(x, 1, 2)


# speedup vs baseline: 1.0788x; 1.0788x over previous
import functools

import jax
import jax.numpy as jnp
from jax.experimental import pallas as pl
from jax.experimental.pallas import tpu as pltpu

_KNOCK = -1e30


def _gat_body(x_ref, w_ref, o_ref,
              xcur_ref, xt_ref, xnext_ref, nrm_ref, s_ref, fbf_ref,
              *, n_pts, k, rows, n_blocks):
    blk = pl.program_id(0)
    i = pl.program_id(1)
    t = pl.program_id(2)

    @pl.when(t == 0)
    def _precompute():
        @pl.when(blk == 0)
        def _():
            xcur_ref[...] = x_ref[0]
        @pl.when(blk > 0)
        def _():
            xcur_ref[...] = xnext_ref[i]
        xv = xcur_ref[...]
        xt_ref[...] = xv.T
        nrm_ref[0, :] = jnp.sum(xv * xv, axis=1)
        f = jnp.where(xv >= 0.0, xv, 0.01 * xv)
        fb16 = f.astype(jnp.bfloat16)
        fbf_ref[...] = fb16
        wb = w_ref[0, 0].astype(jnp.bfloat16).astype(jnp.float32)
        s_ref[0, :] = jnp.sum(fb16.astype(jnp.float32) * wb[None, :], axis=1)

    xr = xcur_ref[pl.ds(t * rows, rows), :]
    xt = xt_ref[...]
    nr = nrm_ref[0, pl.ds(t * rows, rows)]
    s_self = s_ref[0, pl.ds(t * rows, rows)]
    sb = s_ref[0, :][None, :]

    dot = jax.lax.dot_general(xr, xt, (((1,), (0,)), ((), ())),
                              preferred_element_type=jnp.float32)
    pd = 2.0 * dot - nr[:, None] - nrm_ref[0, :][None, :]

    ncolf = -jax.lax.broadcasted_iota(jnp.int32, (rows, n_pts), 1).astype(jnp.float32)
    logits = []
    for _ in range(k):
        mx = jnp.max(pd, axis=1, keepdims=True)
        selneg = jnp.max(jnp.where(pd == mx, ncolf, -jnp.inf),
                         axis=1, keepdims=True)
        hit = ncolf == selneg
        logits.append(jnp.sum(jnp.where(hit, sb, 0.0), axis=1))
        pd = jnp.where(hit, _KNOCK, pd)
    logits.append(s_self)
    lg = jnp.stack(logits, axis=1)

    e = jnp.exp(lg - jnp.max(lg, axis=1, keepdims=True))
    a = e / jnp.sum(e, axis=1, keepdims=True)
    amp = jnp.sum(a.astype(jnp.bfloat16).astype(jnp.float32), axis=1)

    rowg = (jax.lax.broadcasted_iota(jnp.int32, (rows, n_pts), 0)
            + t * rows).astype(jnp.float32)
    grp = jnp.where(pd < -1e29, 1.0, 0.0) + jnp.where(-ncolf == rowg, 1.0, 0.0)
    ssum = jax.lax.dot_general(grp.astype(jnp.bfloat16), fbf_ref[...],
                               (((1,), (0,)), ((), ())),
                               preferred_element_type=jnp.float32)
    y = (amp[:, None] * ssum) / 9.0 + xr
    xnext_ref[i, pl.ds(t * rows, rows), :] = y
    o_ref[0] = y


def kernel(points, W_att):
    b, c, n = points.shape
    n_blocks = W_att.shape[0]
    k = 8
    rows = min(256, n)

    body = functools.partial(_gat_body, n_pts=n, k=k, rows=rows,
                             n_blocks=n_blocks)
    call = pl.pallas_call(
        body,
        grid=(n_blocks, b, n // rows),
        in_specs=[
            pl.BlockSpec((1, n, c), lambda blk, i, t: (i, 0, 0)),
            pl.BlockSpec((1, 1, c), lambda blk, i, t: (blk, 0, 0)),
        ],
        out_specs=pl.BlockSpec((1, rows, c), lambda blk, i, t: (i, t, 0)),
        out_shape=jax.ShapeDtypeStruct((b, n, c), jnp.float32),
        scratch_shapes=[
            pltpu.VMEM((n, c), jnp.float32),
            pltpu.VMEM((c, n), jnp.float32),
            pltpu.VMEM((b, n, c), jnp.float32),
            pltpu.VMEM((1, n), jnp.float32),
            pltpu.VMEM((1, n), jnp.float32),
            pltpu.VMEM((n, c), jnp.bfloat16),
        ],
    )

    x = jnp.swapaxes(points, 1, 2)
    w = W_att[:, :, 0][:, None, :]
    out = call(x, w)
    return jnp.swapaxes(out, 1, 2)
